# baseline (speedup 1.0000x reference)
import functools

import jax
import jax.numpy as jnp
from jax import lax
from jax.experimental import pallas as pl
from jax.experimental.pallas import tpu as pltpu
from jax.experimental.pallas import tpu_sc as plsc

_NC = 2
_NS = 16
_K = 125


def _make_segsum(n, e, dw, stage_tab=False):
    nw = _NC * _NS
    epw = e // nw
    nch = epw // _K
    assert nch % 4 == 0 and nch >= 8
    nps = (n // _NS) & ~7
    rem = n - nps * _NS
    mesh = plsc.VectorSubcoreMesh(core_axis_name="c", subcore_axis_name="s")

    @functools.partial(
        pl.kernel,
        mesh=mesh,
        out_type=jax.ShapeDtypeStruct((_NC, n, dw), jnp.float32),
        scratch_types=[
            [pltpu.VMEM((2, _K), jnp.int32)] * 4,
            [pltpu.VMEM((_K, dw), jnp.float32)] * 2,
            pltpu.VMEM_SHARED((n, dw), jnp.float32),
            (pltpu.VMEM_SHARED((n, dw), jnp.float32) if stage_tab else None),
            [pltpu.SemaphoreType.DMA] * 4,
            [pltpu.SemaphoreType.DMA] * 2,
            [pltpu.SemaphoreType.DMA] * 2,
        ],
        compiler_params=pltpu.CompilerParams(use_tc_tiling_on_sc=False),
    )
    def segsum(tab_hbm, eidx, zeros, out, ebufs, rows, acc, stab, isems,
               gsems, ssems):
        cid = lax.axis_index("c")
        sid = lax.axis_index("s")
        wid = sid * _NC + cid
        pltpu.sync_copy(zeros.at[pl.ds(sid * nps, nps)],
                        acc.at[pl.ds(sid * nps, nps)])
        if stage_tab:
            pltpu.sync_copy(tab_hbm.at[pl.ds(sid * nps, nps)],
                            stab.at[pl.ds(sid * nps, nps)])
        if rem:
            @pl.when(sid == _NS - 1)
            def _():
                pltpu.sync_copy(zeros.at[pl.ds(nps * _NS, rem)],
                                acc.at[pl.ds(nps * _NS, rem)])
                if stage_tab:
                    pltpu.sync_copy(tab_hbm.at[pl.ds(nps * _NS, rem)],
                                    stab.at[pl.ds(nps * _NS, rem)])
        plsc.subcore_barrier()
        tab = stab if stage_tab else tab_hbm

        def idx_start(g, j):
            pltpu.async_copy(eidx.at[0, wid, g], ebufs[j].at[0], isems[j])
            pltpu.async_copy(eidx.at[1, wid, g], ebufs[j].at[1], isems[j])

        def idx_wait(g, j):
            pltpu.make_async_copy(eidx.at[0, wid, g], ebufs[j].at[0],
                                  isems[j]).wait()
            pltpu.make_async_copy(eidx.at[1, wid, g], ebufs[j].at[1],
                                  isems[j]).wait()

        def gat_start(j, b):
            pltpu.async_copy(tab.at[ebufs[j].at[0]], rows[b], gsems[b])

        def gat_wait(j, b):
            pltpu.make_async_copy(tab.at[ebufs[j].at[0]], rows[b],
                                  gsems[b]).wait()

        def sca_start(j, b):
            pltpu.async_copy(rows[b], acc.at[ebufs[j].at[1]], ssems[b],
                             add=True)

        def sca_wait(j, b):
            pltpu.make_async_copy(rows[b], acc.at[ebufs[j].at[1]],
                                  ssems[b]).wait()

        pltpu.sync_copy(eidx.at[0, wid, 0], ebufs[0].at[0])
        pltpu.sync_copy(eidx.at[1, wid, 0], ebufs[0].at[1])
        gat_start(0, 0)
        idx_start(1, 1)
        idx_start(2, 2)
        idx_wait(1, 1)
        gat_start(1, 1)
        idx_start(3, 3)
        gat_wait(0, 0)
        sca_start(0, 0)
        idx_wait(2, 2)
        sca_wait(0, 0)
        gat_start(2, 0)
        idx_start(4, 0)
        gat_wait(1, 1)
        sca_start(1, 1)
        idx_wait(3, 3)
        sca_wait(1, 1)
        gat_start(3, 1)
        idx_start(5, 1)
        gat_wait(2, 0)
        sca_start(2, 0)

        niter = (nch - 4) // 4

        def body(i, carry):
            g0 = i * 4 + 4
            for k in range(4):
                g = g0 + k
                j = k
                b = k % 2
                jp = (k + 3) % 4
                bp = (k + 1) % 2
                idx_wait(g, j)
                sca_wait((k + 2) % 4, b)
                gat_start(j, b)
                if k < 2:
                    idx_start(g + 2, (k + 2) % 4)
                else:
                    @pl.when(i < niter - 1)
                    def _(g=g, k=k):
                        idx_start(g + 2, (k + 2) % 4)
                gat_wait(jp, bp)
                sca_start(jp, bp)
            return carry

        lax.fori_loop(0, niter, body, 0)
        sca_wait(2, 0)
        gat_wait(3, 1)
        sca_start(3, 1)
        sca_wait(3, 1)
        plsc.subcore_barrier()
        pltpu.sync_copy(acc.at[pl.ds(sid * nps, nps)],
                        out.at[cid, pl.ds(sid * nps, nps)])
        if rem:
            @pl.when(sid == _NS - 1)
            def _():
                pltpu.sync_copy(acc.at[pl.ds(nps * _NS, rem)],
                                out.at[cid, pl.ds(nps * _NS, rem)])

    return segsum


def _dense1(P, x, W1l, W1r, b1, W2l, W2r, b2, blk=2000):
    n, d = x.shape
    h_dim = W1l.shape[1]
    c_dim = W2l.shape[1]
    dw = P.shape[2]
    grid = (n // blk,)

    def body(p_ref, x_ref, w1l, w1r, b1r, w2l, w2r, b2r, po, ro, io):
        s = p_ref[0] + p_ref[1]
        deg = jnp.maximum(s[:, d:d + 1], 1.0)
        inv = 1.0 / deg
        agg = s[:, :d] * inv
        h = (jnp.dot(agg, w1l[...], preferred_element_type=jnp.float32)
             + jnp.dot(x_ref[...], w1r[...], preferred_element_type=jnp.float32)
             + b1r[...][None, :])
        h = jnp.maximum(h, 0.0)
        po[...] = jnp.dot(h, w2l[...], preferred_element_type=jnp.float32)
        ro[...] = (jnp.dot(h, w2r[...], preferred_element_type=jnp.float32)
                   + b2r[...][None, :])
        io[...] = jnp.broadcast_to(inv, (blk, 8))

    return pl.pallas_call(
        body,
        grid=grid,
        in_specs=[
            pl.BlockSpec((_NC, blk, dw), lambda i: (0, i, 0)),
            pl.BlockSpec((blk, d), lambda i: (i, 0)),
            pl.BlockSpec((d, h_dim), lambda i: (0, 0)),
            pl.BlockSpec((d, h_dim), lambda i: (0, 0)),
            pl.BlockSpec((h_dim,), lambda i: (0,)),
            pl.BlockSpec((h_dim, c_dim), lambda i: (0, 0)),
            pl.BlockSpec((h_dim, c_dim), lambda i: (0, 0)),
            pl.BlockSpec((c_dim,), lambda i: (0,)),
        ],
        out_specs=[
            pl.BlockSpec((blk, c_dim), lambda i: (i, 0)),
            pl.BlockSpec((blk, c_dim), lambda i: (i, 0)),
            pl.BlockSpec((blk, 8), lambda i: (i, 0)),
        ],
        out_shape=[
            jax.ShapeDtypeStruct((n, c_dim), jnp.float32),
            jax.ShapeDtypeStruct((n, c_dim), jnp.float32),
            jax.ShapeDtypeStruct((n, 8), jnp.float32),
        ],
    )(P, x, W1l, W1r, b1, W2l, W2r, b2)


def _dense2(P2, inv, r, blk=2000):
    n, c_dim = r.shape

    def body(p_ref, inv_ref, r_ref, o_ref):
        s = p_ref[0] + p_ref[1]
        z = s * inv_ref[:, :1] + r_ref[...]
        m = jnp.max(z, axis=1, keepdims=True)
        lse = jnp.log(jnp.sum(jnp.exp(z - m), axis=1, keepdims=True)) + m
        o_ref[...] = z - lse

    return pl.pallas_call(
        body,
        grid=(n // blk,),
        in_specs=[
            pl.BlockSpec((_NC, blk, c_dim), lambda i: (0, i, 0)),
            pl.BlockSpec((blk, 8), lambda i: (i, 0)),
            pl.BlockSpec((blk, c_dim), lambda i: (i, 0)),
        ],
        out_specs=pl.BlockSpec((blk, c_dim), lambda i: (i, 0)),
        out_shape=jax.ShapeDtypeStruct((n, c_dim), jnp.float32),
    )(P2, inv, r)


def kernel(x, edge_index, W1l, W1r, b1, W2l, W2r, b2):
    n, d = x.shape
    e = edge_index.shape[1]
    c_dim = W2l.shape[1]
    nw = _NC * _NS
    nch = e // nw // _K
    eidx = jnp.reshape(edge_index, (2, nw, nch, _K))

    dw1 = d + 16
    xa = jnp.concatenate(
        [x, jnp.ones((n, 1), x.dtype), jnp.zeros((n, dw1 - d - 1), x.dtype)],
        axis=1)
    P1 = _make_segsum(n, e, dw1)(xa, eidx, jnp.zeros((n, dw1), jnp.float32))
    p, r, inv = _dense1(P1, x, W1l, W1r, b1, W2l, W2r, b2)
    P2 = _make_segsum(n, e, c_dim, stage_tab=True)(
        p, eidx, jnp.zeros((n, c_dim), jnp.float32))
    return _dense2(P2, inv, r)


# speedup vs baseline: 1.1760x; 1.1760x over previous
import functools

import jax
import jax.numpy as jnp
from jax import lax
from jax.experimental import pallas as pl
from jax.experimental.pallas import tpu as pltpu
from jax.experimental.pallas import tpu_sc as plsc

_NC = 2
_NS = 16
_K = 125


def _make_segsum(n, e, dw, with_deg=False):
    nw = _NC * _NS
    epw = e // nw
    nch = epw // _K
    assert nch % 4 == 0 and nch >= 8
    nps = (n // _NS) & ~7
    rem = n - nps * _NS
    mesh = plsc.VectorSubcoreMesh(core_axis_name="c", subcore_axis_name="s")

    out_type = [jax.ShapeDtypeStruct((_NC, n, dw), jnp.float32)]
    scratch = [
        [pltpu.VMEM((2, _K), jnp.int32)] * 4,
        [pltpu.VMEM((_K, dw), jnp.float32)] * 2,
        pltpu.VMEM_SHARED((n, dw), jnp.float32),
        [pltpu.SemaphoreType.DMA] * 4,
        [pltpu.SemaphoreType.DMA] * 2,
        [pltpu.SemaphoreType.DMA] * 2,
    ]
    if with_deg:
        out_type.append(jax.ShapeDtypeStruct((_NC, n, 16), jnp.float32))
        scratch += [
            pltpu.VMEM((_K, 16), jnp.float32),
            pltpu.VMEM_SHARED((n, 16), jnp.float32),
            [pltpu.SemaphoreType.DMA] * 2,
        ]

    @functools.partial(
        pl.kernel,
        mesh=mesh,
        out_type=out_type,
        scratch_types=scratch,
        compiler_params=pltpu.CompilerParams(use_tc_tiling_on_sc=False),
    )
    def segsum(*args):
        if with_deg:
            (tab, eidx, zeros, zeros16, out, outd, ebufs, rows, acc,
             isems, gsems, ssems, ones, dacc, dsems) = args
        else:
            (tab, eidx, zeros, out, ebufs, rows, acc,
             isems, gsems, ssems) = args
            zeros16 = outd = ones = dacc = dsems = None
        cid = lax.axis_index("c")
        sid = lax.axis_index("s")
        wid = sid * _NC + cid
        pltpu.sync_copy(zeros.at[pl.ds(sid * nps, nps)],
                        acc.at[pl.ds(sid * nps, nps)])
        if with_deg:
            pltpu.sync_copy(zeros16.at[pl.ds(sid * nps, nps)],
                            dacc.at[pl.ds(sid * nps, nps)])

            def fill_ones(i, carry):
                ones[i, :] = jnp.ones((16,), jnp.float32)
                return carry

            lax.fori_loop(0, _K, fill_ones, 0)
        if rem:
            @pl.when(sid == _NS - 1)
            def _():
                pltpu.sync_copy(zeros.at[pl.ds(nps * _NS, rem)],
                                acc.at[pl.ds(nps * _NS, rem)])
                if with_deg:
                    pltpu.sync_copy(zeros16.at[pl.ds(nps * _NS, rem)],
                                    dacc.at[pl.ds(nps * _NS, rem)])
        plsc.subcore_barrier()

        def idx_start(g, j):
            pltpu.async_copy(eidx.at[0, wid, g], ebufs[j].at[0], isems[j])
            pltpu.async_copy(eidx.at[1, wid, g], ebufs[j].at[1], isems[j])

        def idx_wait(g, j):
            pltpu.make_async_copy(eidx.at[0, wid, g], ebufs[j].at[0],
                                  isems[j]).wait()
            pltpu.make_async_copy(eidx.at[1, wid, g], ebufs[j].at[1],
                                  isems[j]).wait()

        def gat_start(j, b):
            pltpu.async_copy(tab.at[ebufs[j].at[0]], rows[b], gsems[b])

        def gat_wait(j, b):
            pltpu.make_async_copy(tab.at[ebufs[j].at[0]], rows[b],
                                  gsems[b]).wait()

        def sca_start(j, b):
            pltpu.async_copy(rows[b], acc.at[ebufs[j].at[1]], ssems[b],
                             add=True)
            if with_deg:
                pltpu.async_copy(ones, dacc.at[ebufs[j].at[1]], dsems[b],
                                 add=True)

        def sca_wait(j, b):
            pltpu.make_async_copy(rows[b], acc.at[ebufs[j].at[1]],
                                  ssems[b]).wait()
            if with_deg:
                pltpu.make_async_copy(ones, dacc.at[ebufs[j].at[1]],
                                      dsems[b]).wait()

        pltpu.sync_copy(eidx.at[0, wid, 0], ebufs[0].at[0])
        pltpu.sync_copy(eidx.at[1, wid, 0], ebufs[0].at[1])
        gat_start(0, 0)
        idx_start(1, 1)
        idx_start(2, 2)
        idx_wait(1, 1)
        gat_start(1, 1)
        idx_start(3, 3)
        gat_wait(0, 0)
        sca_start(0, 0)
        idx_wait(2, 2)
        sca_wait(0, 0)
        gat_start(2, 0)
        idx_start(4, 0)
        gat_wait(1, 1)
        sca_start(1, 1)
        idx_wait(3, 3)
        sca_wait(1, 1)
        gat_start(3, 1)
        idx_start(5, 1)
        gat_wait(2, 0)
        sca_start(2, 0)

        niter = (nch - 4) // 4

        def body(i, carry):
            g0 = i * 4 + 4
            for k in range(4):
                g = g0 + k
                j = k
                b = k % 2
                jp = (k + 3) % 4
                bp = (k + 1) % 2
                idx_wait(g, j)
                sca_wait((k + 2) % 4, b)
                gat_start(j, b)
                if k < 2:
                    idx_start(g + 2, (k + 2) % 4)
                else:
                    @pl.when(i < niter - 1)
                    def _(g=g, k=k):
                        idx_start(g + 2, (k + 2) % 4)
                gat_wait(jp, bp)
                sca_start(jp, bp)
            return carry

        lax.fori_loop(0, niter, body, 0)
        sca_wait(2, 0)
        gat_wait(3, 1)
        sca_start(3, 1)
        sca_wait(3, 1)
        plsc.subcore_barrier()
        pltpu.sync_copy(acc.at[pl.ds(sid * nps, nps)],
                        out.at[cid, pl.ds(sid * nps, nps)])
        if with_deg:
            pltpu.sync_copy(dacc.at[pl.ds(sid * nps, nps)],
                            outd.at[cid, pl.ds(sid * nps, nps)])
        if rem:
            @pl.when(sid == _NS - 1)
            def _():
                pltpu.sync_copy(acc.at[pl.ds(nps * _NS, rem)],
                                out.at[cid, pl.ds(nps * _NS, rem)])
                if with_deg:
                    pltpu.sync_copy(dacc.at[pl.ds(nps * _NS, rem)],
                                    outd.at[cid, pl.ds(nps * _NS, rem)])

    return segsum


def _dense1(P, Pd, x, W1l, W1r, b1, W2l, W2r, b2, blk=2000):
    n, d = x.shape
    h_dim = W1l.shape[1]
    c_dim = W2l.shape[1]
    dw = P.shape[2]
    grid = (n // blk,)

    def body(p_ref, pd_ref, x_ref, w1l, w1r, b1r, w2l, w2r, b2r, po, ro, io):
        s = p_ref[0] + p_ref[1]
        sd = pd_ref[0] + pd_ref[1]
        deg = jnp.maximum(sd[:, :1], 1.0)
        inv = 1.0 / deg
        agg = s * inv
        h = (jnp.dot(agg, w1l[...], preferred_element_type=jnp.float32)
             + jnp.dot(x_ref[...], w1r[...], preferred_element_type=jnp.float32)
             + b1r[...][None, :])
        h = jnp.maximum(h, 0.0)
        po[...] = jnp.dot(h, w2l[...], preferred_element_type=jnp.float32)
        ro[...] = (jnp.dot(h, w2r[...], preferred_element_type=jnp.float32)
                   + b2r[...][None, :])
        io[...] = jnp.broadcast_to(inv, (blk, 8))

    return pl.pallas_call(
        body,
        grid=grid,
        in_specs=[
            pl.BlockSpec((_NC, blk, dw), lambda i: (0, i, 0)),
            pl.BlockSpec((_NC, blk, 16), lambda i: (0, i, 0)),
            pl.BlockSpec((blk, d), lambda i: (i, 0)),
            pl.BlockSpec((d, h_dim), lambda i: (0, 0)),
            pl.BlockSpec((d, h_dim), lambda i: (0, 0)),
            pl.BlockSpec((h_dim,), lambda i: (0,)),
            pl.BlockSpec((h_dim, c_dim), lambda i: (0, 0)),
            pl.BlockSpec((h_dim, c_dim), lambda i: (0, 0)),
            pl.BlockSpec((c_dim,), lambda i: (0,)),
        ],
        out_specs=[
            pl.BlockSpec((blk, c_dim), lambda i: (i, 0)),
            pl.BlockSpec((blk, c_dim), lambda i: (i, 0)),
            pl.BlockSpec((blk, 8), lambda i: (i, 0)),
        ],
        out_shape=[
            jax.ShapeDtypeStruct((n, c_dim), jnp.float32),
            jax.ShapeDtypeStruct((n, c_dim), jnp.float32),
            jax.ShapeDtypeStruct((n, 8), jnp.float32),
        ],
    )(P, Pd, x, W1l, W1r, b1, W2l, W2r, b2)


def _dense2(P2, inv, r, blk=2000):
    n, c_dim = r.shape

    def body(p_ref, inv_ref, r_ref, o_ref):
        s = p_ref[0] + p_ref[1]
        z = s * inv_ref[:, :1] + r_ref[...]
        m = jnp.max(z, axis=1, keepdims=True)
        lse = jnp.log(jnp.sum(jnp.exp(z - m), axis=1, keepdims=True)) + m
        o_ref[...] = z - lse

    return pl.pallas_call(
        body,
        grid=(n // blk,),
        in_specs=[
            pl.BlockSpec((_NC, blk, c_dim), lambda i: (0, i, 0)),
            pl.BlockSpec((blk, 8), lambda i: (i, 0)),
            pl.BlockSpec((blk, c_dim), lambda i: (i, 0)),
        ],
        out_specs=pl.BlockSpec((blk, c_dim), lambda i: (i, 0)),
        out_shape=jax.ShapeDtypeStruct((n, c_dim), jnp.float32),
    )(P2, inv, r)


def kernel(x, edge_index, W1l, W1r, b1, W2l, W2r, b2):
    n, d = x.shape
    e = edge_index.shape[1]
    c_dim = W2l.shape[1]
    nw = _NC * _NS
    nch = e // nw // _K
    eidx = jnp.reshape(edge_index, (2, nw, nch, _K))

    P1, P1d = _make_segsum(n, e, d, with_deg=True)(
        x, eidx, jnp.zeros((n, d), jnp.float32),
        jnp.zeros((n, 16), jnp.float32))
    p, r, inv = _dense1(P1, P1d, x, W1l, W1r, b1, W2l, W2r, b2)
    (P2,) = _make_segsum(n, e, c_dim)(
        p, eidx, jnp.zeros((n, c_dim), jnp.float32))
    return _dense2(P2, inv, r)


# speedup vs baseline: 1.2176x; 1.0354x over previous
import functools

import jax
import jax.numpy as jnp
from jax import lax
from jax.experimental import pallas as pl
from jax.experimental.pallas import tpu as pltpu
from jax.experimental.pallas import tpu_sc as plsc

_NC = 2
_NS = 16
_K = 125


def _make_segsum(n, e, dw, with_deg=False):
    nw = _NC * _NS
    epw = e // nw
    nch = epw // _K
    assert nch % 4 == 0 and nch >= 8
    nps = (n // _NS) & ~7
    rem = n - nps * _NS
    mesh = plsc.VectorSubcoreMesh(core_axis_name="c", subcore_axis_name="s")

    out_type = [jax.ShapeDtypeStruct((_NC, n, dw), jnp.float32)]
    scratch = [
        [pltpu.VMEM((2, _K), jnp.int32)] * 4,
        [pltpu.VMEM((_K, dw), jnp.float32)] * 2,
        pltpu.VMEM_SHARED((n, dw), jnp.float32),
        [pltpu.SemaphoreType.DMA] * 4,
        [pltpu.SemaphoreType.DMA] * 2,
        [pltpu.SemaphoreType.DMA] * 2,
    ]
    if with_deg:
        out_type.append(jax.ShapeDtypeStruct((_NC, n, 16), jnp.float32))
        scratch += [
            pltpu.VMEM((_K, 16), jnp.float32),
            pltpu.VMEM((_K, 16), jnp.float32),
            pltpu.VMEM_SHARED((n, 16), jnp.float32),
            [pltpu.SemaphoreType.DMA] * 2,
        ]

    @functools.partial(
        pl.kernel,
        mesh=mesh,
        out_type=out_type,
        scratch_types=scratch,
        compiler_params=pltpu.CompilerParams(use_tc_tiling_on_sc=False),
    )
    def segsum(*args):
        if with_deg:
            (tab, eidx, out, outd, ebufs, rows, acc,
             isems, gsems, ssems, ones, z16, dacc, dsems) = args
        else:
            (tab, eidx, out, ebufs, rows, acc,
             isems, gsems, ssems) = args
            outd = ones = z16 = dacc = dsems = None
        cid = lax.axis_index("c")
        sid = lax.axis_index("s")
        wid = sid * _NC + cid

        def fill_zero(i, carry):
            for c in range(dw // 16):
                rows[0][i, pl.ds(c * 16, 16)] = jnp.zeros((16,), jnp.float32)
            return carry

        lax.fori_loop(0, _K, fill_zero, 0)
        if with_deg:
            def fill_ones(i, carry):
                ones[i, :] = jnp.ones((16,), jnp.float32)
                z16[i, :] = jnp.zeros((16,), jnp.float32)
                return carry

            lax.fori_loop(0, _K, fill_ones, 0)

        zb = (_K // 8) * 8

        def clear(base, count):
            off = 0
            while count > 0:
                step = min(zb, count)
                pltpu.sync_copy(rows[0].at[pl.ds(0, step)],
                                acc.at[pl.ds(base + off, step)])
                if with_deg:
                    pltpu.sync_copy(z16.at[pl.ds(0, step)],
                                    dacc.at[pl.ds(base + off, step)])
                off += step
                count -= step

        clear(sid * nps, nps)
        if rem:
            @pl.when(sid == _NS - 1)
            def _():
                clear(nps * _NS, rem)
        plsc.subcore_barrier()

        def idx_start(g, j):
            pltpu.async_copy(eidx.at[0, wid, g], ebufs[j].at[0], isems[j])
            pltpu.async_copy(eidx.at[1, wid, g], ebufs[j].at[1], isems[j])

        def idx_wait(g, j):
            pltpu.make_async_copy(eidx.at[0, wid, g], ebufs[j].at[0],
                                  isems[j]).wait()
            pltpu.make_async_copy(eidx.at[1, wid, g], ebufs[j].at[1],
                                  isems[j]).wait()

        def gat_start(j, b):
            pltpu.async_copy(tab.at[ebufs[j].at[0]], rows[b], gsems[b])

        def gat_wait(j, b):
            pltpu.make_async_copy(tab.at[ebufs[j].at[0]], rows[b],
                                  gsems[b]).wait()

        def sca_start(j, b):
            pltpu.async_copy(rows[b], acc.at[ebufs[j].at[1]], ssems[b],
                             add=True)
            if with_deg:
                pltpu.async_copy(ones, dacc.at[ebufs[j].at[1]], dsems[b],
                                 add=True)

        def sca_wait(j, b):
            pltpu.make_async_copy(rows[b], acc.at[ebufs[j].at[1]],
                                  ssems[b]).wait()
            if with_deg:
                pltpu.make_async_copy(ones, dacc.at[ebufs[j].at[1]],
                                      dsems[b]).wait()

        pltpu.sync_copy(eidx.at[0, wid, 0], ebufs[0].at[0])
        pltpu.sync_copy(eidx.at[1, wid, 0], ebufs[0].at[1])
        gat_start(0, 0)
        idx_start(1, 1)
        idx_start(2, 2)
        idx_wait(1, 1)
        gat_start(1, 1)
        idx_start(3, 3)
        gat_wait(0, 0)
        sca_start(0, 0)
        idx_wait(2, 2)
        sca_wait(0, 0)
        gat_start(2, 0)
        idx_start(4, 0)
        gat_wait(1, 1)
        sca_start(1, 1)
        idx_wait(3, 3)
        sca_wait(1, 1)
        gat_start(3, 1)
        idx_start(5, 1)
        gat_wait(2, 0)
        sca_start(2, 0)

        niter = (nch - 4) // 4

        def body(i, carry):
            g0 = i * 4 + 4
            for k in range(4):
                g = g0 + k
                j = k
                b = k % 2
                jp = (k + 3) % 4
                bp = (k + 1) % 2
                idx_wait(g, j)
                sca_wait((k + 2) % 4, b)
                gat_start(j, b)
                if k < 2:
                    idx_start(g + 2, (k + 2) % 4)
                else:
                    @pl.when(i < niter - 1)
                    def _(g=g, k=k):
                        idx_start(g + 2, (k + 2) % 4)
                gat_wait(jp, bp)
                sca_start(jp, bp)
            return carry

        lax.fori_loop(0, niter, body, 0)
        sca_wait(2, 0)
        gat_wait(3, 1)
        sca_start(3, 1)
        sca_wait(3, 1)
        plsc.subcore_barrier()
        pltpu.sync_copy(acc.at[pl.ds(sid * nps, nps)],
                        out.at[cid, pl.ds(sid * nps, nps)])
        if with_deg:
            pltpu.sync_copy(dacc.at[pl.ds(sid * nps, nps)],
                            outd.at[cid, pl.ds(sid * nps, nps)])
        if rem:
            @pl.when(sid == _NS - 1)
            def _():
                pltpu.sync_copy(acc.at[pl.ds(nps * _NS, rem)],
                                out.at[cid, pl.ds(nps * _NS, rem)])
                if with_deg:
                    pltpu.sync_copy(dacc.at[pl.ds(nps * _NS, rem)],
                                    outd.at[cid, pl.ds(nps * _NS, rem)])

    return segsum


def _dense1(P, Pd, x, W1l, W1r, b1, W2l, W2r, b2, blk=2000):
    n, d = x.shape
    h_dim = W1l.shape[1]
    c_dim = W2l.shape[1]
    dw = P.shape[2]
    grid = (n // blk,)

    def body(p_ref, pd_ref, x_ref, w1l, w1r, b1r, w2l, w2r, b2r, po, ro, io):
        s = p_ref[0] + p_ref[1]
        sd = pd_ref[0] + pd_ref[1]
        deg = jnp.maximum(sd[:, :1], 1.0)
        inv = 1.0 / deg
        agg = s * inv
        h = (jnp.dot(agg, w1l[...], preferred_element_type=jnp.float32)
             + jnp.dot(x_ref[...], w1r[...], preferred_element_type=jnp.float32)
             + b1r[...][None, :])
        h = jnp.maximum(h, 0.0)
        po[...] = jnp.dot(h, w2l[...], preferred_element_type=jnp.float32)
        ro[...] = (jnp.dot(h, w2r[...], preferred_element_type=jnp.float32)
                   + b2r[...][None, :])
        io[...] = jnp.broadcast_to(inv, (blk, 8))

    return pl.pallas_call(
        body,
        grid=grid,
        in_specs=[
            pl.BlockSpec((_NC, blk, dw), lambda i: (0, i, 0)),
            pl.BlockSpec((_NC, blk, 16), lambda i: (0, i, 0)),
            pl.BlockSpec((blk, d), lambda i: (i, 0)),
            pl.BlockSpec((d, h_dim), lambda i: (0, 0)),
            pl.BlockSpec((d, h_dim), lambda i: (0, 0)),
            pl.BlockSpec((h_dim,), lambda i: (0,)),
            pl.BlockSpec((h_dim, c_dim), lambda i: (0, 0)),
            pl.BlockSpec((h_dim, c_dim), lambda i: (0, 0)),
            pl.BlockSpec((c_dim,), lambda i: (0,)),
        ],
        out_specs=[
            pl.BlockSpec((blk, c_dim), lambda i: (i, 0)),
            pl.BlockSpec((blk, c_dim), lambda i: (i, 0)),
            pl.BlockSpec((blk, 8), lambda i: (i, 0)),
        ],
        out_shape=[
            jax.ShapeDtypeStruct((n, c_dim), jnp.float32),
            jax.ShapeDtypeStruct((n, c_dim), jnp.float32),
            jax.ShapeDtypeStruct((n, 8), jnp.float32),
        ],
    )(P, Pd, x, W1l, W1r, b1, W2l, W2r, b2)


def _dense2(P2, inv, r, blk=2000):
    n, c_dim = r.shape

    def body(p_ref, inv_ref, r_ref, o_ref):
        s = p_ref[0] + p_ref[1]
        z = s * inv_ref[:, :1] + r_ref[...]
        m = jnp.max(z, axis=1, keepdims=True)
        lse = jnp.log(jnp.sum(jnp.exp(z - m), axis=1, keepdims=True)) + m
        o_ref[...] = z - lse

    return pl.pallas_call(
        body,
        grid=(n // blk,),
        in_specs=[
            pl.BlockSpec((_NC, blk, c_dim), lambda i: (0, i, 0)),
            pl.BlockSpec((blk, 8), lambda i: (i, 0)),
            pl.BlockSpec((blk, c_dim), lambda i: (i, 0)),
        ],
        out_specs=pl.BlockSpec((blk, c_dim), lambda i: (i, 0)),
        out_shape=jax.ShapeDtypeStruct((n, c_dim), jnp.float32),
    )(P2, inv, r)


def kernel(x, edge_index, W1l, W1r, b1, W2l, W2r, b2):
    n, d = x.shape
    e = edge_index.shape[1]
    c_dim = W2l.shape[1]
    nw = _NC * _NS
    nch = e // nw // _K
    eidx = jnp.reshape(edge_index, (2, nw, nch, _K))

    P1, P1d = _make_segsum(n, e, d, with_deg=True)(x, eidx)
    p, r, inv = _dense1(P1, P1d, x, W1l, W1r, b1, W2l, W2r, b2)
    (P2,) = _make_segsum(n, e, c_dim)(p, eidx)
    return _dense2(P2, inv, r)
